# baseline (speedup 1.0000x reference)
import functools

import jax
import jax.numpy as jnp
from jax import lax
from jax.experimental import pallas as pl
from jax.experimental.pallas import tpu as pltpu
from jax.experimental.pallas import tpu_sc as plsc

NUM_WORKERS = 32
CHUNK_B = 1
NBUF = 4
IDX_SLICE = 128
PD = 128


def _build_kernel(B, T, D):
    rows_per_w = (B * T) // NUM_WORKERS
    chunk_rows = CHUNK_B * T
    chunks_per_w = rows_per_w // chunk_rows
    n_full, rem = divmod(chunk_rows, IDX_SLICE)
    mesh = plsc.VectorSubcoreMesh(core_axis_name="c", subcore_axis_name="s")

    scratch = [pltpu.VMEM((rows_per_w,), jnp.int32)]
    scratch += [pltpu.VMEM((chunk_rows, PD), jnp.float32) for _ in range(NBUF)]
    scratch += [pltpu.SemaphoreType.DMA for _ in range(3 * NBUF)]

    @functools.partial(
        pl.kernel,
        mesh=mesh,
        compiler_params=pltpu.CompilerParams(use_tc_tiling_on_sc=True),
        out_type=jax.ShapeDtypeStruct((B * T, PD), jnp.float32),
        scratch_types=scratch,
    )
    def emb_kernel(idx_hbm, table_hbm, pos2_hbm, out_hbm, idx_v, *rest):
        bufs = rest[:NBUF]
        pres = rest[NBUF:2 * NBUF]
        gsem = rest[2 * NBUF:3 * NBUF]
        osem = rest[3 * NBUF:4 * NBUF]
        wid = lax.axis_index("s") * 2 + lax.axis_index("c")
        base = wid * rows_per_w
        pltpu.sync_copy(idx_hbm.at[pl.ds(base, rows_per_w)], idx_v)

        def prefill(c):
            p = c % NBUF
            return pltpu.async_copy(pos2_hbm, bufs[p], pres[p])

        def gathers(c):
            p = c % NBUF
            cps = []
            r0 = c * chunk_rows
            for j in range(n_full):
                cps.append(pltpu.async_copy(
                    table_hbm.at[idx_v.at[pl.ds(r0 + j * IDX_SLICE, IDX_SLICE)]],
                    bufs[p].at[pl.ds(j * IDX_SLICE, IDX_SLICE)],
                    gsem[p], add=True))
            if rem:
                cps.append(pltpu.async_copy(
                    table_hbm.at[idx_v.at[pl.ds(r0 + n_full * IDX_SLICE, rem)]],
                    bufs[p].at[pl.ds(n_full * IDX_SLICE, rem)],
                    gsem[p], add=True))
            return cps

        def writeback(c):
            p = c % NBUF
            return pltpu.async_copy(
                bufs[p], out_hbm.at[pl.ds(base + c * chunk_rows, chunk_rows)],
                osem[p])

        pre_cp = [None] * chunks_per_w
        out_cp = [None] * chunks_per_w
        pre_cp[0] = prefill(0)
        pre_cp[1] = prefill(1)
        for c in range(chunks_per_w):
            if c + 2 < chunks_per_w:
                if c >= 2:
                    out_cp[c - 2].wait()
                pre_cp[c + 2] = prefill(c + 2)
            pre_cp[c].wait()
            g_cp = gathers(c)
            for cp in g_cp:
                cp.wait()
            out_cp[c] = writeback(c)
        out_cp[chunks_per_w - 2].wait()
        out_cp[chunks_per_w - 1].wait()

    return emb_kernel


def kernel(x, token_table, pos_table):
    B, T = x.shape
    V, D = token_table.shape
    flat_idx = (x.reshape(B * T) >> 1).astype(jnp.int32)
    table_pairs = token_table.reshape(V // 2, 2 * D)
    pos2 = jnp.tile(jnp.pad(pos_table, ((0, 0), (0, PD - D))), (CHUNK_B, 1))
    out = _build_kernel(B, T, D)(flat_idx, table_pairs, pos2)
    return out.reshape(B, T, PD)[:, :, :D]


# speedup vs baseline: 1.1495x; 1.1495x over previous
import functools

import jax
import jax.numpy as jnp
from jax import lax
from jax.experimental import pallas as pl
from jax.experimental.pallas import tpu as pltpu
from jax.experimental.pallas import tpu_sc as plsc

NUM_WORKERS = 32
CHUNK_B = 2
NBUF = 4
IDX_SLICE = 128
PD = 128


def _build_kernel(B, T, D):
    rows_per_w = (B * T) // NUM_WORKERS
    chunk_rows = CHUNK_B * T
    chunks_per_w = rows_per_w // chunk_rows
    n_full, rem = divmod(chunk_rows, IDX_SLICE)
    mesh = plsc.VectorSubcoreMesh(core_axis_name="c", subcore_axis_name="s")

    scratch = [pltpu.VMEM((rows_per_w,), jnp.int32)]
    scratch += [pltpu.VMEM((chunk_rows, D), jnp.float32) for _ in range(NBUF)]
    scratch += [pltpu.SemaphoreType.DMA for _ in range(3 * NBUF)]

    @functools.partial(
        pl.kernel,
        mesh=mesh,
        compiler_params=pltpu.CompilerParams(use_tc_tiling_on_sc=False),
        out_type=jax.ShapeDtypeStruct((B * T, PD), jnp.float32),
        scratch_types=scratch,
    )
    def emb_kernel(idx_hbm, table_hbm, pos2_hbm, out_hbm, idx_v, *rest):
        bufs = rest[:NBUF]
        pres = rest[NBUF:2 * NBUF]
        gsem = rest[2 * NBUF:3 * NBUF]
        osem = rest[3 * NBUF:4 * NBUF]
        wid = lax.axis_index("s") * 2 + lax.axis_index("c")
        base = wid * rows_per_w
        pltpu.sync_copy(idx_hbm.at[pl.ds(base, rows_per_w)], idx_v)

        def prefill(c):
            p = c % NBUF
            return pltpu.async_copy(pos2_hbm, bufs[p], pres[p])

        def gathers(c):
            p = c % NBUF
            cps = []
            r0 = c * chunk_rows
            for j in range(n_full):
                cps.append(pltpu.async_copy(
                    table_hbm.at[idx_v.at[pl.ds(r0 + j * IDX_SLICE, IDX_SLICE)]],
                    bufs[p].at[pl.ds(j * IDX_SLICE, IDX_SLICE)],
                    gsem[p], add=True))
            if rem:
                cps.append(pltpu.async_copy(
                    table_hbm.at[idx_v.at[pl.ds(r0 + n_full * IDX_SLICE, rem)]],
                    bufs[p].at[pl.ds(n_full * IDX_SLICE, rem)],
                    gsem[p], add=True))
            return cps

        def writeback(c):
            p = c % NBUF
            return pltpu.async_copy(
                bufs[p],
                out_hbm.at[pl.ds(base + c * chunk_rows, chunk_rows), pl.ds(0, D)],
                osem[p])

        pre_cp = [None] * chunks_per_w
        out_cp = [None] * chunks_per_w
        pre_cp[0] = prefill(0)
        pre_cp[1] = prefill(1)
        for c in range(chunks_per_w):
            if c + 2 < chunks_per_w:
                if c >= 2:
                    out_cp[c - 2].wait()
                pre_cp[c + 2] = prefill(c + 2)
            pre_cp[c].wait()
            g_cp = gathers(c)
            for cp in g_cp:
                cp.wait()
            out_cp[c] = writeback(c)
        out_cp[chunks_per_w - 2].wait()
        out_cp[chunks_per_w - 1].wait()

    return emb_kernel


def kernel(x, token_table, pos_table):
    B, T = x.shape
    V, D = token_table.shape
    flat_idx = x.reshape(B * T).astype(jnp.int32)
    pos2 = jnp.tile(pos_table, (CHUNK_B, 1))
    out = _build_kernel(B, T, D)(flat_idx, token_table, pos2)
    return out.reshape(B, T, PD)[:, :, :D]


# speedup vs baseline: 1.2821x; 1.1154x over previous
import functools

import jax
import jax.numpy as jnp
from jax import lax
from jax.experimental import pallas as pl
from jax.experimental.pallas import tpu as pltpu
from jax.experimental.pallas import tpu_sc as plsc

NUM_WORKERS = 32
CHUNK_B = 2
NBUF = 4
IDX_SLICE = 128
PD = 128


def _build_kernel(B, T, D):
    rows_per_w = (B * T) // NUM_WORKERS
    chunk_rows = CHUNK_B * T
    chunks_per_w = rows_per_w // chunk_rows
    n_full, rem = divmod(chunk_rows, IDX_SLICE)
    mesh = plsc.VectorSubcoreMesh(core_axis_name="c", subcore_axis_name="s")

    scratch = [pltpu.VMEM((rows_per_w,), jnp.int32)]
    scratch += [pltpu.VMEM((chunk_rows, D), jnp.float32) for _ in range(NBUF)]
    scratch += [pltpu.SemaphoreType.DMA for _ in range(3 * NBUF)]
    scratch += [pltpu.VMEM_SHARED((chunk_rows, D), jnp.float32)]

    @functools.partial(
        pl.kernel,
        mesh=mesh,
        compiler_params=pltpu.CompilerParams(use_tc_tiling_on_sc=False),
        out_type=jax.ShapeDtypeStruct((B * T, PD), jnp.float32),
        scratch_types=scratch,
    )
    def emb_kernel(idx_hbm, table_hbm, pos2_hbm, out_hbm, idx_v, *rest):
        bufs = rest[:NBUF]
        pres = rest[NBUF:2 * NBUF]
        gsem = rest[2 * NBUF:3 * NBUF]
        osem = rest[3 * NBUF:4 * NBUF]
        pos_sh = rest[4 * NBUF]
        sid = lax.axis_index("s")
        wid = sid * 2 + lax.axis_index("c")
        base = wid * rows_per_w

        @pl.when(sid == 0)
        def _stage_pos():
            pltpu.sync_copy(pos2_hbm, pos_sh)

        pltpu.sync_copy(idx_hbm.at[pl.ds(base, rows_per_w)], idx_v)
        plsc.subcore_barrier()

        def prefill(c):
            p = c % NBUF
            return pltpu.async_copy(pos_sh, bufs[p], pres[p])

        def gathers(c):
            p = c % NBUF
            cps = []
            r0 = c * chunk_rows
            for j in range(n_full):
                cps.append(pltpu.async_copy(
                    table_hbm.at[idx_v.at[pl.ds(r0 + j * IDX_SLICE, IDX_SLICE)]],
                    bufs[p].at[pl.ds(j * IDX_SLICE, IDX_SLICE)],
                    gsem[p], add=True))
            if rem:
                cps.append(pltpu.async_copy(
                    table_hbm.at[idx_v.at[pl.ds(r0 + n_full * IDX_SLICE, rem)]],
                    bufs[p].at[pl.ds(n_full * IDX_SLICE, rem)],
                    gsem[p], add=True))
            return cps

        def writeback(c):
            p = c % NBUF
            return pltpu.async_copy(
                bufs[p],
                out_hbm.at[pl.ds(base + c * chunk_rows, chunk_rows), pl.ds(0, D)],
                osem[p])

        pre_cp = [None] * chunks_per_w
        out_cp = [None] * chunks_per_w
        pre_cp[0] = prefill(0)
        pre_cp[1] = prefill(1)
        for c in range(chunks_per_w):
            if c + 2 < chunks_per_w:
                if c >= 2:
                    out_cp[c - 2].wait()
                pre_cp[c + 2] = prefill(c + 2)
            pre_cp[c].wait()
            g_cp = gathers(c)
            for cp in g_cp:
                cp.wait()
            out_cp[c] = writeback(c)
        out_cp[chunks_per_w - 2].wait()
        out_cp[chunks_per_w - 1].wait()

    return emb_kernel


def kernel(x, token_table, pos_table):
    B, T = x.shape
    V, D = token_table.shape
    flat_idx = x.reshape(B * T).astype(jnp.int32)
    pos2 = jnp.tile(pos_table, (CHUNK_B, 1))
    out = _build_kernel(B, T, D)(flat_idx, token_table, pos2)
    return out.reshape(B, T, PD)[:, :, :D]
